# baseline (speedup 1.0000x reference)
import functools
import math

import jax
import jax.numpy as jnp
import numpy as np
from jax import lax
from jax.experimental import pallas as pl
from jax.experimental.pallas import tpu as pltpu
from jax.experimental.pallas import tpu_sc as plsc

NUM_NODES = 160
SEQ_LEN = 128
BATCH = 8
TOPK = 20
_ROWS_PER_W = 8
_NWORKERS = NUM_NODES // _ROWS_PER_W

_DINV = np.float32(np.float32(20.0) ** np.float32(-0.5))
_NORM = np.float32(_DINV * _DINV)
_NEG = np.float32(-3.0e38)

_G = np.asarray(
    jax.random.gumbel(jax.random.key(42), (NUM_NODES * NUM_NODES, 2), jnp.float32)
)
_GD = (_G[:, 1] - _G[:, 0]).reshape(BATCH, NUM_NODES, TOPK)


def _cos_kernel(emb_ref, o_ref):
    emb = emb_ref[...]
    dot = jax.lax.dot_general(
        emb, emb, (((1,), (1,)), ((), ())), preferred_element_type=jnp.float32
    )
    row_i = jax.lax.broadcasted_iota(jnp.int32, (NUM_NODES, NUM_NODES), 0)
    col_i = jax.lax.broadcasted_iota(jnp.int32, (NUM_NODES, NUM_NODES), 1)
    eye = (row_i == col_i).astype(jnp.float32)
    n2_col = jnp.sum(emb * emb, axis=1, keepdims=True)
    n2_row = jnp.max(eye * n2_col, axis=0, keepdims=True)
    o_ref[...] = dot / (jnp.sqrt(n2_col) * jnp.sqrt(n2_row))


def _adj_sc_kernel(cos_hbm, dl_hbm, gd_hbm, a_hbm, cos_v, dl_v, gd_v, a_v,
                   tf_v, ti_v):
    c = lax.axis_index("c")
    s = lax.axis_index("s")
    wid = s * 2 + c
    base = wid * _ROWS_PER_W

    @pl.when(wid < _NWORKERS)
    def _work():
        _adj_sc_body(base, cos_hbm, dl_hbm, gd_hbm, a_hbm, cos_v, dl_v, gd_v, a_v,
                     tf_v, ti_v)


def _allmax_f32(v, tmp_ref, iota):
    for sh in (8, 4, 2, 1):
        tmp_ref[...] = v
        v = jnp.maximum(v, plsc.load_gather(tmp_ref, [iota ^ sh]))
    return v


def _allmin_i32(v, tmp_ref, iota):
    for sh in (8, 4, 2, 1):
        tmp_ref[...] = v
        v = jnp.minimum(v, plsc.load_gather(tmp_ref, [iota ^ sh]))
    return v


def _adj_sc_body(base, cos_hbm, dl_hbm, gd_hbm, a_hbm, cos_v, dl_v, gd_v, a_v,
                 tf_v, ti_v):
    pltpu.sync_copy(cos_hbm.at[pl.ds(base, _ROWS_PER_W)], cos_v)
    pltpu.sync_copy(dl_hbm, dl_v)
    pltpu.sync_copy(gd_hbm, gd_v)

    iota = lax.iota(jnp.int32, 16)
    mask8 = iota < BATCH
    lane0 = iota == 0
    zero16 = jnp.zeros((16,), jnp.float32)

    for r in range(_ROWS_PER_W):
        for b in range(BATCH):
            for cc in range(NUM_NODES // 16):
                a_v[r, b, pl.ds(cc * 16, 16)] = zero16

    for r in range(_ROWS_PER_W):
        i = base + r

        def _step(t, _):
            mx = jnp.full((16,), _NEG, jnp.float32)
            for cc in range(NUM_NODES // 16):
                mx = jnp.maximum(mx, cos_v[r, pl.ds(cc * 16, 16)])
            m = _allmax_f32(mx, tf_v, iota)
            jm = jnp.full((16,), np.int32(NUM_NODES), jnp.int32)
            for cc in range(NUM_NODES // 16):
                v = cos_v[r, pl.ds(cc * 16, 16)]
                jm = jnp.minimum(
                    jm, jnp.where(v == m, iota + np.int32(cc * 16), np.int32(NUM_NODES))
                )
            jvec = _allmin_i32(jm, ti_v, iota)

            ivec = jnp.full((16,), i, jnp.int32)
            tvec = jnp.full((16,), t, jnp.int32)
            dlv = plsc.load_gather(dl_v, [iota, ivec, tvec], mask=mask8)
            gdv = plsc.load_gather(gd_v, [iota, ivec, tvec], mask=mask8)
            val = jnp.where(dlv >= gdv, _NORM, np.float32(0.0))
            rvec = jnp.full((16,), r, jnp.int32)
            plsc.store_scatter(a_v, [rvec, iota, jvec], val, mask=mask8)
            plsc.store_scatter(
                cos_v,
                [rvec, jvec],
                jnp.full((16,), _NEG, jnp.float32),
                mask=lane0,
            )
            return 0

        lax.fori_loop(0, TOPK, _step, 0)

    pltpu.sync_copy(a_v, a_hbm.at[pl.ds(base, _ROWS_PER_W)])


def _mm_kernel(x_ref, w_ref, a_ref, b_ref, o_ref):
    xb = x_ref[0]
    w = w_ref[...]
    h = jax.lax.dot_general(
        w, xb, (((0,), (0,)), ((), ())), preferred_element_type=jnp.float32
    )
    bias = b_ref[...]
    for bn in range(BATCH):
        hb = h[:, bn * NUM_NODES : (bn + 1) * NUM_NODES]
        ab = a_ref[:, bn, :]
        ob = jax.lax.dot_general(
            hb, ab, (((1,), (1,)), ((), ())), preferred_element_type=jnp.float32
        )
        o_ref[0, :, bn * NUM_NODES : (bn + 1) * NUM_NODES] = ob + bias


@functools.partial(
    pl.kernel,
    out_type=jax.ShapeDtypeStruct((NUM_NODES, BATCH, NUM_NODES), jnp.float32),
    mesh=plsc.VectorSubcoreMesh(core_axis_name="c", subcore_axis_name="s"),
    compiler_params=pltpu.CompilerParams(
        needs_layout_passes=False, use_tc_tiling_on_sc=False
    ),
    scratch_types=[
        pltpu.VMEM((_ROWS_PER_W, NUM_NODES), jnp.float32),
        pltpu.VMEM((BATCH, NUM_NODES, TOPK), jnp.float32),
        pltpu.VMEM((BATCH, NUM_NODES, TOPK), jnp.float32),
        pltpu.VMEM((_ROWS_PER_W, BATCH, NUM_NODES), jnp.float32),
        pltpu.VMEM((16,), jnp.float32),
        pltpu.VMEM((16,), jnp.int32),
    ],
)
def _adj_sc(*refs):
    _adj_sc_kernel(*refs)


def kernel(x, emb_table, weight, bias, logits):
    n_total = BATCH * NUM_NODES
    dl = (logits[:, 0] - logits[:, 1]).reshape(BATCH, NUM_NODES, TOPK)

    cos = pl.pallas_call(
        _cos_kernel,
        out_shape=jax.ShapeDtypeStruct((NUM_NODES, NUM_NODES), jnp.float32),
    )(emb_table)

    a = _adj_sc(cos, dl, jnp.asarray(_GD))

    out = pl.pallas_call(
        _mm_kernel,
        grid=(BATCH,),
        in_specs=[
            pl.BlockSpec((1, SEQ_LEN, n_total), lambda i: (i, 0, 0)),
            pl.BlockSpec((SEQ_LEN, SEQ_LEN), lambda i: (0, 0)),
            pl.BlockSpec((NUM_NODES, BATCH, NUM_NODES), lambda i: (0, 0, 0)),
            pl.BlockSpec((SEQ_LEN, 1), lambda i: (0, 0)),
        ],
        out_specs=pl.BlockSpec((1, SEQ_LEN, n_total), lambda i: (i, 0, 0)),
        out_shape=jax.ShapeDtypeStruct((BATCH, SEQ_LEN, n_total), jnp.float32),
    )(x, weight, a, bias.reshape(SEQ_LEN, 1))
    return out
